# baseline (speedup 1.0000x reference)
import dataclasses
import functools

import jax
import jax.numpy as jnp
from jax import lax
from jax.experimental import pallas as pl
from jax.experimental.pallas import tpu as pltpu
from jax.experimental.pallas import tpu_sc as plsc

_NUM_WORKERS = 32
_CHUNK = 2048
_GSUB = 128


def _sc_grid_sample(quad, grid_flat, *, H, W, N, Ho, Wo):
    P = N * Ho * Wo
    HOWO = Ho * Wo
    imgs_per_worker = N // _NUM_WORKERS
    chunks_per_img = HOWO // _CHUNK
    sx = (W - 1) / 2.0
    sy = (H - 1) / 2.0

    mesh = plsc.VectorSubcoreMesh(core_axis_name="c", subcore_axis_name="s")

    cp = pltpu.CompilerParams()
    for fld, val in (("needs_layout_passes", False),
                     ("use_tc_tiling_on_sc", False)):
        if fld in pltpu.CompilerParams.__dataclass_fields__:
            cp = dataclasses.replace(cp, **{fld: val})

    @functools.partial(
        pl.kernel,
        compiler_params=cp,
        out_type=jax.ShapeDtypeStruct((N * 2 * HOWO,), jnp.float32),
        mesh=mesh,
        scratch_types=[
            pltpu.VMEM((2 * _CHUNK,), jnp.float32),
            pltpu.VMEM((_CHUNK,), jnp.int32),
            pltpu.VMEM((_CHUNK,), jnp.float32),
            pltpu.VMEM((_CHUNK,), jnp.float32),
            pltpu.VMEM((_CHUNK, 8), jnp.float32),
            pltpu.VMEM((_CHUNK,), jnp.float32),
            pltpu.VMEM((_CHUNK,), jnp.float32),
            pltpu.SemaphoreType.DMA,
        ],
    )
    def sc_kernel(quad_hbm, grid_hbm, out_hbm,
                  g_v, idx_v, wx_v, wy_v, rows_v, out0_v, out1_v, sem):
        cid = lax.axis_index("c")
        sid = lax.axis_index("s")
        wid = sid * 2 + cid

        lane = lax.iota(jnp.int32, 16)

        @pl.loop(0, imgs_per_worker)
        def _(im):
            n = wid * imgs_per_worker + im

            @pl.loop(0, chunks_per_img)
            def _(j):
                p_off = n * HOWO + j * _CHUNK
                pltpu.sync_copy(grid_hbm.at[pl.ds(2 * p_off, 2 * _CHUNK)], g_v)

                @pl.loop(0, _CHUNK, step=16)
                def _(i):
                    gpos = (i + lane) * 2
                    gx = plsc.load_gather(g_v, [gpos])
                    gy = plsc.load_gather(g_v, [gpos + 1])
                    x = gx * sx + sx
                    y = gy * sy + sy
                    x = jnp.minimum(jnp.maximum(x, 0.0), W - 1.0)
                    y = jnp.minimum(jnp.maximum(y, 0.0), H - 1.0)
                    xi = x.astype(jnp.int32)
                    yi = y.astype(jnp.int32)
                    wx = x - xi.astype(jnp.float32)
                    wy = y - yi.astype(jnp.float32)
                    idx_v[pl.ds(i, 16)] = yi * W + xi
                    wx_v[pl.ds(i, 16)] = wx
                    wy_v[pl.ds(i, 16)] = wy

                copies = []
                for k in range(_CHUNK // _GSUB):
                    copies.append(pltpu.async_copy(
                        quad_hbm.at[idx_v.at[pl.ds(k * _GSUB, _GSUB)]],
                        rows_v.at[pl.ds(k * _GSUB, _GSUB)],
                        sem))
                for c in copies:
                    c.wait()

                @pl.loop(0, _CHUNK, step=16)
                def _(i):
                    p = i + lane
                    wx = wx_v[pl.ds(i, 16)]
                    wy = wy_v[pl.ds(i, 16)]
                    taps = [plsc.load_gather(
                        rows_v, [p, jnp.full((16,), t, jnp.int32)])
                        for t in range(8)]
                    t0 = taps[0] + wx * (taps[1] - taps[0])
                    b0 = taps[2] + wx * (taps[3] - taps[2])
                    t1 = taps[4] + wx * (taps[5] - taps[4])
                    b1 = taps[6] + wx * (taps[7] - taps[6])
                    out0_v[pl.ds(i, 16)] = t0 + wy * (b0 - t0)
                    out1_v[pl.ds(i, 16)] = t1 + wy * (b1 - t1)

                o0 = n * 2 * HOWO + j * _CHUNK
                pltpu.sync_copy(out0_v, out_hbm.at[pl.ds(o0, _CHUNK)])
                pltpu.sync_copy(out1_v, out_hbm.at[pl.ds(o0 + HOWO, _CHUNK)])

    return sc_kernel(quad, grid_flat)


def kernel(grid, inp):
    N, Ho, Wo, _ = grid.shape
    _, C, H, W = inp.shape
    assert C == 2 and N % _NUM_WORKERS == 0 and (Ho * Wo) % _CHUNK == 0

    flat = inp[0].reshape(C, H * W)

    def shifts(f):
        s1 = jnp.concatenate([f[1:], f[-1:]])
        sw = jnp.concatenate([f[W:], f[-W:]])
        sw1 = jnp.concatenate([sw[1:], sw[-1:]])
        return [f, s1, sw, sw1]

    quad = jnp.stack(shifts(flat[0]) + shifts(flat[1]), axis=1)
    grid_flat = grid.reshape(-1)

    out_flat = _sc_grid_sample(quad, grid_flat, H=H, W=W, N=N, Ho=Ho, Wo=Wo)
    return out_flat.reshape(N, C, Ho, Wo)


# speedup vs baseline: 1.0943x; 1.0943x over previous
import dataclasses
import functools

import jax
import jax.numpy as jnp
from jax import lax
from jax.experimental import pallas as pl
from jax.experimental.pallas import tpu as pltpu
from jax.experimental.pallas import tpu_sc as plsc

_NUM_WORKERS = 32
_CHUNK = 2048
_GSUB = 128


def _sc_grid_sample(tab0, tab1, grid_flat, *, H, W, N, Ho, Wo):
    HOWO = Ho * Wo
    HW = H * W
    imgs_per_worker = N // _NUM_WORKERS
    chunks_per_img = HOWO // _CHUNK
    sx = (W - 1) / 2.0
    sy = (H - 1) / 2.0

    mesh = plsc.VectorSubcoreMesh(core_axis_name="c", subcore_axis_name="s")

    cp = pltpu.CompilerParams()
    for fld, val in (("needs_layout_passes", False),
                     ("use_tc_tiling_on_sc", False)):
        if fld in pltpu.CompilerParams.__dataclass_fields__:
            cp = dataclasses.replace(cp, **{fld: val})

    @functools.partial(
        pl.kernel,
        compiler_params=cp,
        out_type=jax.ShapeDtypeStruct((N * 2 * HOWO,), jnp.float32),
        mesh=mesh,
        scratch_types=[
            pltpu.VMEM((2 * _CHUNK,), jnp.float32),
            pltpu.VMEM((_CHUNK,), jnp.int32),
            pltpu.VMEM((_CHUNK,), jnp.int32),
            pltpu.VMEM((_CHUNK,), jnp.float32),
            pltpu.VMEM((_CHUNK,), jnp.float32),
            pltpu.VMEM((_CHUNK,), jnp.int32),
            pltpu.VMEM((_CHUNK,), jnp.int32),
            pltpu.VMEM((_CHUNK,), jnp.int32),
            pltpu.VMEM((_CHUNK,), jnp.int32),
            pltpu.VMEM((_CHUNK,), jnp.float32),
            pltpu.VMEM((_CHUNK,), jnp.float32),
            pltpu.SemaphoreType.DMA,
        ],
    )
    def sc_kernel(tab0_hbm, tab1_hbm, grid_hbm, out_hbm,
                  g_v, idx_v, idxw_v, wx_v, wy_v,
                  t0_v, b0_v, t1_v, b1_v, out0_v, out1_v, sem):
        cid = lax.axis_index("c")
        sid = lax.axis_index("s")
        wid = sid * 2 + cid

        lane = lax.iota(jnp.int32, 16)
        himask = jnp.full((16,), -65536, jnp.int32)

        @pl.loop(0, imgs_per_worker)
        def _(im):
            n = wid * imgs_per_worker + im

            @pl.loop(0, chunks_per_img)
            def _(j):
                p_off = n * HOWO + j * _CHUNK
                pltpu.sync_copy(grid_hbm.at[pl.ds(2 * p_off, 2 * _CHUNK)], g_v)

                @pl.loop(0, _CHUNK, step=16)
                def _(i):
                    gpos = (i + lane) * 2
                    gx = plsc.load_gather(g_v, [gpos])
                    gy = plsc.load_gather(g_v, [gpos + 1])
                    x = gx * sx + sx
                    y = gy * sy + sy
                    x = jnp.minimum(jnp.maximum(x, 0.0), W - 1.0)
                    y = jnp.minimum(jnp.maximum(y, 0.0), H - 1.0)
                    xi = x.astype(jnp.int32)
                    yi = y.astype(jnp.int32)
                    wx = x - xi.astype(jnp.float32)
                    wy = y - yi.astype(jnp.float32)
                    idx = yi * W + xi
                    idx_v[pl.ds(i, 16)] = idx
                    idxw_v[pl.ds(i, 16)] = jnp.minimum(idx + W, HW - 1)
                    wx_v[pl.ds(i, 16)] = wx
                    wy_v[pl.ds(i, 16)] = wy

                copies = []
                for k in range(_CHUNK // _GSUB):
                    s = pl.ds(k * _GSUB, _GSUB)
                    copies.append(pltpu.async_copy(
                        tab0_hbm.at[idx_v.at[s]], t0_v.at[s], sem))
                    copies.append(pltpu.async_copy(
                        tab0_hbm.at[idxw_v.at[s]], b0_v.at[s], sem))
                    copies.append(pltpu.async_copy(
                        tab1_hbm.at[idx_v.at[s]], t1_v.at[s], sem))
                    copies.append(pltpu.async_copy(
                        tab1_hbm.at[idxw_v.at[s]], b1_v.at[s], sem))
                for c in copies:
                    c.wait()

                @pl.loop(0, _CHUNK, step=16)
                def _(i):
                    s = pl.ds(i, 16)
                    wx = wx_v[s]
                    wy = wy_v[s]
                    wt0 = t0_v[s]
                    wb0 = b0_v[s]
                    wt1 = t1_v[s]
                    wb1 = b1_v[s]
                    v00 = plsc.bitcast(wt0 << 16, jnp.float32)
                    v01 = plsc.bitcast(wt0 & himask, jnp.float32)
                    v10 = plsc.bitcast(wb0 << 16, jnp.float32)
                    v11 = plsc.bitcast(wb0 & himask, jnp.float32)
                    u00 = plsc.bitcast(wt1 << 16, jnp.float32)
                    u01 = plsc.bitcast(wt1 & himask, jnp.float32)
                    u10 = plsc.bitcast(wb1 << 16, jnp.float32)
                    u11 = plsc.bitcast(wb1 & himask, jnp.float32)
                    top0 = v00 + wx * (v01 - v00)
                    bot0 = v10 + wx * (v11 - v10)
                    top1 = u00 + wx * (u01 - u00)
                    bot1 = u10 + wx * (u11 - u10)
                    out0_v[s] = top0 + wy * (bot0 - top0)
                    out1_v[s] = top1 + wy * (bot1 - top1)

                o0 = n * 2 * HOWO + j * _CHUNK
                pltpu.sync_copy(out0_v, out_hbm.at[pl.ds(o0, _CHUNK)])
                pltpu.sync_copy(out1_v, out_hbm.at[pl.ds(o0 + HOWO, _CHUNK)])

    return sc_kernel(tab0, tab1, grid_flat)


def _pack_pairs(plane_flat):
    lo = lax.bitcast_convert_type(
        plane_flat.astype(jnp.bfloat16), jnp.uint16).astype(jnp.uint32)
    hi = jnp.concatenate([lo[1:], lo[-1:]])
    return (lo | (hi << 16)).astype(jnp.int32)


def kernel(grid, inp):
    N, Ho, Wo, _ = grid.shape
    _, C, H, W = inp.shape
    assert C == 2 and N % _NUM_WORKERS == 0 and (Ho * Wo) % _CHUNK == 0

    flat = inp[0].reshape(C, H * W)
    tab0 = _pack_pairs(flat[0])
    tab1 = _pack_pairs(flat[1])
    grid_flat = grid.reshape(-1)

    out_flat = _sc_grid_sample(tab0, tab1, grid_flat,
                               H=H, W=W, N=N, Ho=Ho, Wo=Wo)
    return out_flat.reshape(N, C, Ho, Wo)


# speedup vs baseline: 8.0226x; 7.3316x over previous
import dataclasses
import functools

import jax
import jax.numpy as jnp
from jax import lax
from jax.experimental import pallas as pl
from jax.experimental.pallas import tpu as pltpu
from jax.experimental.pallas import tpu_sc as plsc

_NUM_WORKERS = 32
_CHUNK = 2048
_GSUB = 128


def _sc_grid_sample(tab0, tab1, grid_lin, *, H, W, N, Ho, Wo):
    HOWO = Ho * Wo
    HW = H * W
    imgs_per_worker = N // _NUM_WORKERS
    chunks_per_img = HOWO // _CHUNK
    nch = imgs_per_worker * chunks_per_img
    sx = (W - 1) / 2.0
    sy = (H - 1) / 2.0

    mesh = plsc.VectorSubcoreMesh(core_axis_name="c", subcore_axis_name="s")

    cp = pltpu.CompilerParams()
    for fld, val in (("needs_layout_passes", False),
                     ("use_tc_tiling_on_sc", False)):
        if fld in pltpu.CompilerParams.__dataclass_fields__:
            cp = dataclasses.replace(cp, **{fld: val})

    vm = pltpu.VMEM
    scratch = []
    for _ in range(2):
        scratch += [
            vm((2 * _CHUNK,), jnp.float32),
            vm((_CHUNK,), jnp.int32),
            vm((_CHUNK,), jnp.int32),
            vm((_CHUNK,), jnp.float32),
            vm((_CHUNK,), jnp.float32),
            vm((_CHUNK,), jnp.int32),
            vm((_CHUNK,), jnp.int32),
            vm((_CHUNK,), jnp.int32),
            vm((_CHUNK,), jnp.int32),
            vm((_CHUNK,), jnp.float32),
            vm((_CHUNK,), jnp.float32),
            pltpu.SemaphoreType.DMA,
            pltpu.SemaphoreType.DMA,
            pltpu.SemaphoreType.DMA,
        ]

    @functools.partial(
        pl.kernel,
        compiler_params=cp,
        out_type=jax.ShapeDtypeStruct((N * 2 * HOWO,), jnp.float32),
        mesh=mesh,
        scratch_types=scratch,
    )
    def sc_kernel(tab0_hbm, tab1_hbm, grid_hbm, out_hbm, *bufs):
        cid = lax.axis_index("c")
        sid = lax.axis_index("s")
        wid = sid * 2 + cid

        names = ("g", "idx", "idxr", "wx", "wy", "l0", "r0", "l1", "r1",
                 "o0", "o1", "gsem", "rsem", "osem")
        B = [dict(zip(names, bufs[:14])), dict(zip(names, bufs[14:]))]

        def p_off(t):
            n = wid * imgs_per_worker + t // chunks_per_img
            j = lax.rem(t, chunks_per_img)
            return n * HOWO + j * _CHUNK

        def o_off(t):
            n = wid * imgs_per_worker + t // chunks_per_img
            j = lax.rem(t, chunks_per_img)
            return n * 2 * HOWO + j * _CHUNK

        def start_grid(t, b):
            pltpu.async_copy(grid_hbm.at[pl.ds(2 * p_off(t), 2 * _CHUNK)],
                             B[b]["g"], B[b]["gsem"])

        def wait_grid(b):
            pltpu.make_async_copy(grid_hbm.at[pl.ds(0, 2 * _CHUNK)],
                                  B[b]["g"], B[b]["gsem"]).wait()

        def compute(b):
            g_v, idx_v, idxr_v = B[b]["g"], B[b]["idx"], B[b]["idxr"]
            wx_v, wy_v = B[b]["wx"], B[b]["wy"]

            @pl.loop(0, _CHUNK, step=16)
            def _(i):
                base = (i // 128) * 256 + lax.rem(i, 128)
                gx = g_v[pl.ds(base, 16)]
                gy = g_v[pl.ds(base + 128, 16)]
                x = gx * sx + sx
                y = gy * sy + sy
                x = jnp.minimum(jnp.maximum(x, 0.0), W - 1.0)
                y = jnp.minimum(jnp.maximum(y, 0.0), H - 1.0)
                xi = x.astype(jnp.int32)
                yi = y.astype(jnp.int32)
                idx = xi * H + yi
                idx_v[pl.ds(i, 16)] = idx
                idxr_v[pl.ds(i, 16)] = jnp.minimum(idx + H, HW - 1)
                wx_v[pl.ds(i, 16)] = x - xi.astype(jnp.float32)
                wy_v[pl.ds(i, 16)] = y - yi.astype(jnp.float32)

        def gather_copies(b):
            d = B[b]
            for k in range(_CHUNK // _GSUB):
                s = pl.ds(k * _GSUB, _GSUB)
                yield pltpu.make_async_copy(
                    tab0_hbm.at[d["idx"].at[s]], d["l0"].at[s], d["rsem"])
                yield pltpu.make_async_copy(
                    tab0_hbm.at[d["idxr"].at[s]], d["r0"].at[s], d["rsem"])
                yield pltpu.make_async_copy(
                    tab1_hbm.at[d["idx"].at[s]], d["l1"].at[s], d["rsem"])
                yield pltpu.make_async_copy(
                    tab1_hbm.at[d["idxr"].at[s]], d["r1"].at[s], d["rsem"])

        def fire(b):
            for c in gather_copies(b):
                c.start()

        def drain(b):
            for c in gather_copies(b):
                c.wait()

        himask = jnp.full((16,), -65536, jnp.int32)

        def combine(b):
            d = B[b]
            l0_v, r0_v, l1_v, r1_v = d["l0"], d["r0"], d["l1"], d["r1"]
            wx_v, wy_v, out0_v, out1_v = d["wx"], d["wy"], d["o0"], d["o1"]

            @pl.loop(0, _CHUNK, step=16)
            def _(i):
                s = pl.ds(i, 16)
                wx = wx_v[s]
                wy = wy_v[s]
                wl0 = l0_v[s]
                wr0 = r0_v[s]
                wl1 = l1_v[s]
                wr1 = r1_v[s]
                v00 = plsc.bitcast(wl0 << 16, jnp.float32)
                v10 = plsc.bitcast(wl0 & himask, jnp.float32)
                v01 = plsc.bitcast(wr0 << 16, jnp.float32)
                v11 = plsc.bitcast(wr0 & himask, jnp.float32)
                u00 = plsc.bitcast(wl1 << 16, jnp.float32)
                u10 = plsc.bitcast(wl1 & himask, jnp.float32)
                u01 = plsc.bitcast(wr1 << 16, jnp.float32)
                u11 = plsc.bitcast(wr1 & himask, jnp.float32)
                cl0 = v00 + wy * (v10 - v00)
                cr0 = v01 + wy * (v11 - v01)
                cl1 = u00 + wy * (u10 - u00)
                cr1 = u01 + wy * (u11 - u01)
                out0_v[s] = cl0 + wx * (cr0 - cl0)
                out1_v[s] = cl1 + wx * (cr1 - cl1)

        def out_copies(t, b):
            d = B[b]
            o0 = o_off(t)
            yield pltpu.make_async_copy(
                d["o0"], out_hbm.at[pl.ds(o0, _CHUNK)], d["osem"])
            yield pltpu.make_async_copy(
                d["o1"], out_hbm.at[pl.ds(o0 + HOWO, _CHUNK)], d["osem"])

        def drain_out(b):
            for c in out_copies(0, b):
                c.wait()

        def stage_a(t, b):
            wait_grid(b)
            compute(b)
            fire(b)

        def stage_b(t, b):
            drain(b)

            @pl.when(t >= 2)
            def _():
                drain_out(b)

            combine(b)
            for c in out_copies(t, b):
                c.start()

        start_grid(0, 0)
        stage_a(0, 0)
        start_grid(1, 1)

        @pl.loop(0, nch, step=2)
        def _(t):
            stage_a(t + 1, 1)

            @pl.when(t + 2 < nch)
            def _():
                start_grid(t + 2, 0)

            stage_b(t, 0)

            @pl.when(t + 2 < nch)
            def _():
                stage_a(t + 2, 0)
                start_grid(t + 3, 1)

            stage_b(t + 1, 1)

        drain_out(0)
        drain_out(1)

    return sc_kernel(tab0, tab1, grid_lin)


def _pack_vert_pairs(plane_cm):
    lo = lax.bitcast_convert_type(
        plane_cm.astype(jnp.bfloat16), jnp.uint16).astype(jnp.uint32)
    hi = jnp.concatenate([lo[1:], lo[-1:]])
    return (lo | (hi << 16)).astype(jnp.int32)


def kernel(grid, inp):
    N, Ho, Wo, _ = grid.shape
    _, C, H, W = inp.shape
    assert C == 2 and N % _NUM_WORKERS == 0 and (Ho * Wo) % _CHUNK == 0
    assert Wo >= 128 and Wo % 128 == 0 and _CHUNK % 128 == 0

    tab0 = _pack_vert_pairs(inp[0, 0].T.reshape(-1))
    tab1 = _pack_vert_pairs(inp[0, 1].T.reshape(-1))
    grid_lin = grid.transpose(0, 1, 3, 2).reshape(-1)

    out_flat = _sc_grid_sample(tab0, tab1, grid_lin,
                               H=H, W=W, N=N, Ho=Ho, Wo=Wo)
    return out_flat.reshape(N, C, Ho, Wo)
